# Initial kernel scaffold; baseline (speedup 1.0000x reference)
import jax
import jax.numpy as jnp
from jax.experimental import pallas as pl


def kernel(idx, token_embedding_table):
    raise NotImplementedError("write your pallas kernel here")


# speedup vs baseline: 1.6194x; 1.6194x over previous
import functools

import jax
import jax.numpy as jnp
from jax import lax
from jax.experimental import pallas as pl
from jax.experimental.pallas import tpu as pltpu
from jax.experimental.pallas import tpu_sc as plsc

_VOCAB = 8192
_ROWS = 8192
_NC, _NS = 2, 16
_NW = _NC * _NS
_RPW = _ROWS // _NW
_CH = 4
_NCHUNK = _RPW // _CH


def _sc_gather(idx2d, table):
    mesh = plsc.VectorSubcoreMesh(core_axis_name="c", subcore_axis_name="s")

    @functools.partial(
        pl.kernel,
        mesh=mesh,
        out_type=jax.ShapeDtypeStruct((_ROWS, _VOCAB), jnp.float32),
        scratch_types=[
            pltpu.VMEM((_NCHUNK, _CH), jnp.int32),
            pltpu.VMEM((_CH, _VOCAB), jnp.float32),
            pltpu.SemaphoreType.DMA,
        ],
    )
    def k(idx_hbm, table_hbm, out_hbm, idx_v, buf, sem):
        wid = lax.axis_index("s") * _NC + lax.axis_index("c")
        pltpu.sync_copy(idx_hbm.at[pl.ds(wid * _NCHUNK, _NCHUNK)], idx_v)

        def body(g, carry):
            pltpu.async_copy(table_hbm.at[idx_v.at[g]], buf, sem).wait()
            pltpu.sync_copy(buf, out_hbm.at[pl.ds(wid * _RPW + g * _CH, _CH)])
            return carry

        lax.fori_loop(0, _NCHUNK, body, 0)

    return k(idx2d, table)


def kernel(idx, token_embedding_table):
    b, t = idx.shape
    idx2d = idx.reshape(_ROWS // _CH, _CH)
    out = _sc_gather(idx2d, token_embedding_table)
    return out.reshape(b, t, _VOCAB)


# speedup vs baseline: 1.9418x; 1.1991x over previous
import functools

import jax
import jax.numpy as jnp
from jax import lax
from jax.experimental import pallas as pl
from jax.experimental.pallas import tpu as pltpu
from jax.experimental.pallas import tpu_sc as plsc

_VOCAB = 8192
_ROWS = 8192
_NC, _NS = 2, 16
_NW = _NC * _NS
_RPW = _ROWS // _NW
_CH = 4
_NCHUNK = _RPW // _CH


def _sc_gather(idx2d, table):
    mesh = plsc.VectorSubcoreMesh(core_axis_name="c", subcore_axis_name="s")

    @functools.partial(
        pl.kernel,
        mesh=mesh,
        out_type=jax.ShapeDtypeStruct((_ROWS, _VOCAB), jnp.float32),
        scratch_types=[
            pltpu.VMEM((_NCHUNK, _CH), jnp.int32),
            pltpu.VMEM((_CH, _VOCAB), jnp.float32),
            pltpu.VMEM((_CH, _VOCAB), jnp.float32),
            pltpu.SemaphoreType.DMA,
            pltpu.SemaphoreType.DMA,
            pltpu.SemaphoreType.DMA,
            pltpu.SemaphoreType.DMA,
        ],
    )
    def k(idx_hbm, table_hbm, out_hbm, idx_v, buf0, buf1, g0, g1, s0, s1):
        wid = lax.axis_index("s") * _NC + lax.axis_index("c")
        base = wid * _RPW
        pltpu.sync_copy(idx_hbm.at[pl.ds(wid * _NCHUNK, _NCHUNK)], idx_v)
        bufs = (buf0, buf1)
        gsem = (g0, g1)
        ssem = (s0, s1)

        def pair(p, carry):
            for b in range(2):
                c = p * 2 + b

                @pl.when(c >= 2)
                def _():
                    pltpu.make_async_copy(
                        bufs[b], out_hbm.at[pl.ds(0, _CH)], ssem[b]
                    ).wait()

                pltpu.async_copy(
                    table_hbm.at[idx_v.at[c]], bufs[b], gsem[b]
                ).wait()
                pltpu.async_copy(
                    bufs[b], out_hbm.at[pl.ds(base + c * _CH, _CH)], ssem[b]
                )
            return carry

        lax.fori_loop(0, _NCHUNK // 2, pair, 0)
        for b in range(2):
            pltpu.make_async_copy(
                bufs[b], out_hbm.at[pl.ds(0, _CH)], ssem[b]
            ).wait()

    return k(idx2d, table)


def kernel(idx, token_embedding_table):
    b, t = idx.shape
    idx2d = idx.reshape(_ROWS // _CH, _CH)
    out = _sc_gather(idx2d, token_embedding_table)
    return out.reshape(b, t, _VOCAB)


# speedup vs baseline: 1.9615x; 1.0102x over previous
import functools

import jax
import jax.numpy as jnp
from jax import lax
from jax.experimental import pallas as pl
from jax.experimental.pallas import tpu as pltpu
from jax.experimental.pallas import tpu_sc as plsc

_VOCAB = 8192
_ROWS = 8192
_NC, _NS = 2, 16
_NW = _NC * _NS
_RPW = _ROWS // _NW
_CH = 2
_NBUF = 4
_NCHUNK = _RPW // _CH


def _sc_gather(idx2d, table):
    mesh = plsc.VectorSubcoreMesh(core_axis_name="c", subcore_axis_name="s")

    @functools.partial(
        pl.kernel,
        mesh=mesh,
        out_type=jax.ShapeDtypeStruct((_ROWS, _VOCAB), jnp.float32),
        scratch_types=(
            [pltpu.VMEM((_NCHUNK, _CH), jnp.int32)]
            + [pltpu.VMEM((_CH, _VOCAB), jnp.float32)] * _NBUF
            + [pltpu.SemaphoreType.DMA] * (2 * _NBUF)
        ),
    )
    def k(idx_hbm, table_hbm, out_hbm, idx_v, *rest):
        bufs = rest[:_NBUF]
        gsem = rest[_NBUF:2 * _NBUF]
        ssem = rest[2 * _NBUF:]
        wid = lax.axis_index("s") * _NC + lax.axis_index("c")
        base = wid * _RPW
        pltpu.sync_copy(idx_hbm.at[pl.ds(wid * _NCHUNK, _NCHUNK)], idx_v)

        pltpu.async_copy(table_hbm.at[idx_v.at[0]], bufs[0], gsem[0])

        def step(p, carry):
            for b in range(_NBUF):
                c = p * _NBUF + b
                b1 = (b + 1) % _NBUF

                @pl.when(c + 1 < _NCHUNK)
                def _():
                    @pl.when(c + 1 >= _NBUF)
                    def _():
                        pltpu.make_async_copy(
                            bufs[b1], out_hbm.at[pl.ds(0, _CH)], ssem[b1]
                        ).wait()

                    pltpu.async_copy(
                        table_hbm.at[idx_v.at[c + 1]], bufs[b1], gsem[b1]
                    )

                pltpu.make_async_copy(
                    table_hbm.at[idx_v.at[c]], bufs[b], gsem[b]
                ).wait()
                pltpu.async_copy(
                    bufs[b], out_hbm.at[pl.ds(base + c * _CH, _CH)], ssem[b]
                )
            return carry

        lax.fori_loop(0, _NCHUNK // _NBUF, step, 0)
        for b in range(_NBUF):
            pltpu.make_async_copy(
                bufs[b], out_hbm.at[pl.ds(0, _CH)], ssem[b]
            ).wait()

    return k(idx2d, table)


def kernel(idx, token_embedding_table):
    b, t = idx.shape
    idx2d = idx.reshape(_ROWS // _CH, _CH)
    out = _sc_gather(idx2d, token_embedding_table)
    return out.reshape(b, t, _VOCAB)
